# Initial kernel scaffold; baseline (speedup 1.0000x reference)
import jax
import jax.numpy as jnp
from jax.experimental import pallas as pl


def kernel(Tempo, Bar, Position, Token, Duration, tempo_table, bar_table, pos_table, token_table, dur_table, W_dec, b_dec):
    raise NotImplementedError("write your pallas kernel here")


# speedup vs baseline: 6.0293x; 6.0293x over previous
import functools

import jax
import jax.numpy as jnp
from jax import lax
from jax.experimental import pallas as pl
from jax.experimental.pallas import tpu as pltpu
from jax.experimental.pallas import tpu_sc as plsc

D = 64
B, L = 64, 2048
N = B * L
OFFS = (0, 64, 192, 320, 576)
NROWS = 704

NC, NS = 2, 16
NW = NC * NS
TPW = N // NW
IG = 128
CHUNK = 256
NG = CHUNK // IG
NCHUNK = TPW // CHUNK


def _project_body(tt, bt, pt, kt, dt, w, b, out_ref):
    bias = b[0, :]
    out_ref[0:64, :] = jnp.dot(tt[...], w[0:64, :],
                               preferred_element_type=jnp.float32) + bias
    out_ref[64:192, :] = jnp.dot(bt[...], w[64:128, :],
                                 preferred_element_type=jnp.float32)
    out_ref[192:320, :] = jnp.dot(pt[...], w[128:192, :],
                                  preferred_element_type=jnp.float32)
    out_ref[320:576, :] = jnp.dot(kt[...], w[192:256, :],
                                  preferred_element_type=jnp.float32)
    out_ref[576:704, :] = jnp.dot(dt[...], w[256:320, :],
                                  preferred_element_type=jnp.float32)


def _project(tt, bt, pt, kt, dt, w, b):
    return pl.pallas_call(
        _project_body,
        out_shape=jax.ShapeDtypeStruct((NROWS, D), jnp.float32),
    )(tt, bt, pt, kt, dt, w, b.reshape(1, D))


def _lookup_body(ctab, i0, i1, i2, i3, i4, out,
                 v0, v1, v2, v3, v4, r0, r1, r2, r3, r4, acc, sem):
    cid = lax.axis_index("c")
    sid = lax.axis_index("s")
    wid = sid * NC + cid
    row0 = wid * (TPW // IG)

    def chunk_body(k, carry):
        irow = row0 + k * NG
        base = irow * IG
        pltpu.sync_copy(i0.at[pl.ds(irow, NG)], v0)
        pltpu.sync_copy(i1.at[pl.ds(irow, NG)], v1)
        pltpu.sync_copy(i2.at[pl.ds(irow, NG)], v2)
        pltpu.sync_copy(i3.at[pl.ds(irow, NG)], v3)
        pltpu.sync_copy(i4.at[pl.ds(irow, NG)], v4)

        def off_body(j, c):
            g = j // (IG // 16)
            col = (j % (IG // 16)) * 16
            sl = pl.ds(col, 16)
            v1[g, sl] = v1[g, sl] + OFFS[1]
            v2[g, sl] = v2[g, sl] + OFFS[2]
            v3[g, sl] = v3[g, sl] + OFFS[3]
            v4[g, sl] = v4[g, sl] + OFFS[4]
            return c
        lax.fori_loop(0, NG * (IG // 16), off_body, 0)

        cps = []
        for g in range(NG):
            dst = pl.ds(g * IG, IG)
            cps.append(pltpu.async_copy(ctab.at[v0.at[g]], r0.at[dst], sem))
            cps.append(pltpu.async_copy(ctab.at[v1.at[g]], r1.at[dst], sem))
            cps.append(pltpu.async_copy(ctab.at[v2.at[g]], r2.at[dst], sem))
            cps.append(pltpu.async_copy(ctab.at[v3.at[g]], r3.at[dst], sem))
            cps.append(pltpu.async_copy(ctab.at[v4.at[g]], r4.at[dst], sem))
        for cp in cps:
            cp.wait()

        def acc_body(t, c):
            for cc in range(D // 16):
                sl = pl.ds(cc * 16, 16)
                acc[t, sl] = (r0[t, sl] + r1[t, sl] + r2[t, sl]
                              + r3[t, sl] + r4[t, sl])
            return c
        lax.fori_loop(0, CHUNK, acc_body, 0)

        pltpu.sync_copy(acc, out.at[pl.ds(base, CHUNK)])
        return carry

    lax.fori_loop(0, NCHUNK, chunk_body, 0)


@functools.partial(jax.jit, static_argnums=())
def _lookup(ctab, i0, i1, i2, i3, i4):
    mesh = plsc.VectorSubcoreMesh(core_axis_name="c", subcore_axis_name="s")
    f = pl.kernel(
        _lookup_body,
        out_type=jax.ShapeDtypeStruct((N, D), jnp.float32),
        mesh=mesh,
        scratch_types=[
            pltpu.VMEM((NG, IG), jnp.int32),
            pltpu.VMEM((NG, IG), jnp.int32),
            pltpu.VMEM((NG, IG), jnp.int32),
            pltpu.VMEM((NG, IG), jnp.int32),
            pltpu.VMEM((NG, IG), jnp.int32),
            pltpu.VMEM((CHUNK, D), jnp.float32),
            pltpu.VMEM((CHUNK, D), jnp.float32),
            pltpu.VMEM((CHUNK, D), jnp.float32),
            pltpu.VMEM((CHUNK, D), jnp.float32),
            pltpu.VMEM((CHUNK, D), jnp.float32),
            pltpu.VMEM((CHUNK, D), jnp.float32),
            pltpu.SemaphoreType.DMA,
        ],
        compiler_params=pltpu.CompilerParams(use_tc_tiling_on_sc=False),
    )
    return f(ctab, i0, i1, i2, i3, i4)


def kernel(Tempo, Bar, Position, Token, Duration, tempo_table, bar_table,
           pos_table, token_table, dur_table, W_dec, b_dec):
    ctab = _project(tempo_table, bar_table, pos_table, token_table,
                    dur_table, W_dec, b_dec)
    shp = (N // IG, IG)
    out = _lookup(
        ctab,
        Tempo.reshape(shp), Bar.reshape(shp), Position.reshape(shp),
        Token.reshape(shp), Duration.reshape(shp),
    )
    return out.reshape(B, L, D)


# speedup vs baseline: 10.2647x; 1.7025x over previous
import jax
import jax.numpy as jnp
from jax import lax
from jax.experimental import pallas as pl
from jax.experimental.pallas import tpu as pltpu
from jax.experimental.pallas import tpu_sc as plsc

D = 64
B, L = 64, 2048
N = B * L
N_T, N_B, N_P, N_K, N_D = 64, 128, 128, 256, 128

NC, NS = 2, 16
NW = NC * NS
TPW = N // NW
IG = 128
CHUNK = 256
NG = CHUNK // IG
NCHUNK = TPW // CHUNK


def _project_body(tt, bt, pt, kt, dt, w, b, otb, opd, otok):
    bias = b[0, :]
    p_t = jnp.dot(tt[...], w[0:64, :], preferred_element_type=jnp.float32) + bias
    p_b = jnp.dot(bt[...], w[64:128, :], preferred_element_type=jnp.float32)
    p_p = jnp.dot(pt[...], w[128:192, :], preferred_element_type=jnp.float32)
    p_k = jnp.dot(kt[...], w[192:256, :], preferred_element_type=jnp.float32)
    p_d = jnp.dot(dt[...], w[256:320, :], preferred_element_type=jnp.float32)
    otb[...] = p_t[:, None, :] + p_b[None, :, :]
    opd[...] = p_p[:, None, :] + p_d[None, :, :]
    otok[...] = p_k


def _project(tt, bt, pt, kt, dt, w, b):
    return pl.pallas_call(
        _project_body,
        out_shape=[
            jax.ShapeDtypeStruct((N_T, N_B, D), jnp.float32),
            jax.ShapeDtypeStruct((N_P, N_D, D), jnp.float32),
            jax.ShapeDtypeStruct((N_K, D), jnp.float32),
        ],
    )(tt, bt, pt, kt, dt, w, b.reshape(1, D))


def _lookup_body(ttb, tpd, ttok, iall, out,
                 vi0, vi1, ci0, ci1, ra0, ra1, rb0, rb1, rc0, rc1,
                 sidx, sg, so):
    cid = lax.axis_index("c")
    sid = lax.axis_index("s")
    wid = sid * NC + cid
    row0 = wid * (TPW // IG)
    tok0 = wid * TPW
    vi = (vi0, vi1)
    ci = (ci0, ci1)
    ra = (ra0, ra1)
    rb = (rb0, rb1)
    rc = (rc0, rc1)

    def idx_cp(k):
        return pltpu.make_async_copy(
            iall.at[pl.ds(row0 + k * NG, NG)], vi[k & 1], sidx)

    def gather_cps(k):
        p = k & 1
        cps = []
        for g in range(NG):
            dst = pl.ds(g * IG, IG)
            cps.append(pltpu.make_async_copy(
                ttb.at[ci[p].at[g, 0]], ra[p].at[dst], sg))
            cps.append(pltpu.make_async_copy(
                tpd.at[ci[p].at[g, 1]], rb[p].at[dst], sg))
            cps.append(pltpu.make_async_copy(
                ttok.at[vi[p].at[g, 3]], rc[p].at[dst], sg))
        return cps

    def out_cp(k):
        return pltpu.make_async_copy(
            ra[k & 1], out.at[pl.ds(tok0 + k * CHUNK, CHUNK)], so)

    def combine(k):
        p = k & 1
        vip, cip = vi[p], ci[p]

        def cb(j, c):
            g = j // (IG // 16)
            col = (j % (IG // 16)) * 16
            sl = pl.ds(col, 16)
            cip[g, 0, sl] = vip[g, 0, sl] * N_B + vip[g, 1, sl]
            cip[g, 1, sl] = vip[g, 2, sl] * N_D + vip[g, 4, sl]
            return c
        lax.fori_loop(0, NG * (IG // 16), cb, 0)

    def accum(k):
        p = k & 1
        rap, rbp, rcp = ra[p], rb[p], rc[p]

        def ab(t, c):
            for cc in range(D // 16):
                sl = pl.ds(cc * 16, 16)
                rap[t, sl] = rap[t, sl] + rbp[t, sl] + rcp[t, sl]
            return c
        lax.fori_loop(0, CHUNK, ab, 0)

    idx_cp(0).start()
    idx_cp(0).wait()
    combine(0)
    for cp in gather_cps(0):
        cp.start()
    if NCHUNK > 1:
        idx_cp(1).start()

    for k in range(NCHUNK):
        if k + 1 < NCHUNK:
            idx_cp(k + 1).wait()
            combine(k + 1)
        for cp in gather_cps(k):
            cp.wait()
        if k >= 1:
            out_cp(k - 1).wait()
        if k + 1 < NCHUNK:
            for cp in gather_cps(k + 1):
                cp.start()
            if k + 2 < NCHUNK:
                idx_cp(k + 2).start()
        accum(k)
        out_cp(k).start()
    out_cp(NCHUNK - 1).wait()


def _lookup(ttb, tpd, ttok, iall):
    mesh = plsc.VectorSubcoreMesh(core_axis_name="c", subcore_axis_name="s")
    f = pl.kernel(
        _lookup_body,
        out_type=jax.ShapeDtypeStruct((N, D), jnp.float32),
        mesh=mesh,
        scratch_types=[
            pltpu.VMEM((NG, 5, IG), jnp.int32),
            pltpu.VMEM((NG, 5, IG), jnp.int32),
            pltpu.VMEM((NG, 2, IG), jnp.int32),
            pltpu.VMEM((NG, 2, IG), jnp.int32),
            pltpu.VMEM((CHUNK, D), jnp.float32),
            pltpu.VMEM((CHUNK, D), jnp.float32),
            pltpu.VMEM((CHUNK, D), jnp.float32),
            pltpu.VMEM((CHUNK, D), jnp.float32),
            pltpu.VMEM((CHUNK, D), jnp.float32),
            pltpu.VMEM((CHUNK, D), jnp.float32),
            pltpu.SemaphoreType.DMA,
            pltpu.SemaphoreType.DMA,
            pltpu.SemaphoreType.DMA,
        ],
        compiler_params=pltpu.CompilerParams(use_tc_tiling_on_sc=False),
    )
    return f(ttb, tpd, ttok, iall)


def kernel(Tempo, Bar, Position, Token, Duration, tempo_table, bar_table,
           pos_table, token_table, dur_table, W_dec, b_dec):
    ttb3, tpd3, ttok = _project(tempo_table, bar_table, pos_table,
                                token_table, dur_table, W_dec, b_dec)
    ttb = ttb3.reshape(N_T * N_B, D)
    tpd = tpd3.reshape(N_P * N_D, D)
    iall = (jnp.stack([Tempo.reshape(N), Bar.reshape(N), Position.reshape(N),
                       Token.reshape(N), Duration.reshape(N)])
            .reshape(5, N // IG, IG).transpose(1, 0, 2))
    out = _lookup(ttb, tpd, ttok, iall)
    return out.reshape(B, L, D)
